# baseline (speedup 1.0000x reference)
import functools
import math

import jax
import jax.numpy as jnp
from jax import lax
from jax.experimental import pallas as pl
from jax.experimental.pallas import tpu as pltpu
from jax.experimental.pallas import tpu_sc as plsc

N = 384
D = 300
E = 12288
NN = N * N
_INV_SQRT_D = 1.0 / math.sqrt(float(D))

_NSUB = 16
_EPT = E // _NSUB
_CH = _EPT // 128
_SLICE = NN // _NSUB
_ROWS = N // _NSUB


def _sc_counts(ei1, ei2):
    mesh = plsc.VectorSubcoreMesh(core_axis_name="c", subcore_axis_name="s")

    @functools.partial(
        pl.kernel,
        mesh=mesh,
        out_type=jax.ShapeDtypeStruct((2, NN), jnp.float32),
        scratch_types=[
            pltpu.VMEM((_EPT,), jnp.int32),
            pltpu.VMEM((_EPT,), jnp.int32),
            pltpu.VMEM((_EPT,), jnp.int32),
            pltpu.VMEM((_EPT,), jnp.int32),
            pltpu.VMEM((_CH, 128), jnp.int32),
            pltpu.VMEM((128,), jnp.float32),
            pltpu.VMEM((_SLICE,), jnp.float32),
            pltpu.VMEM_SHARED((NN,), jnp.float32),
            pltpu.SemaphoreType.DMA,
            pltpu.SemaphoreType.DMA,
            pltpu.SemaphoreType.DMA,
        ],
    )
    def counts_kernel(ei1_hbm, ei2_hbm, out_hbm, src1_v, dst1_v, src2_v,
                      dst2_v, flat_v, ones_v, stage_v, acc_sh, sem_e, sem_z,
                      sem_s):
        c = lax.axis_index("c")
        s = lax.axis_index("s")
        base = s * _EPT

        e_cps = [
            pltpu.async_copy(ei1_hbm.at[0, pl.ds(base, _EPT)], src1_v, sem_e),
            pltpu.async_copy(ei1_hbm.at[1, pl.ds(base, _EPT)], dst1_v, sem_e),
            pltpu.async_copy(ei2_hbm.at[0, pl.ds(base, _EPT)], src2_v, sem_e),
            pltpu.async_copy(ei2_hbm.at[1, pl.ds(base, _EPT)], dst2_v, sem_e),
        ]

        zeros16 = jnp.zeros((16,), jnp.float32)

        def zero_body(i, carry):
            for u in range(16):
                stage_v[pl.ds(i * 256 + u * 16, 16)] = zeros16
            return carry

        lax.fori_loop(0, _SLICE // 256, zero_body, 0)
        for i in range(8):
            ones_v[pl.ds(i * 16, 16)] = jnp.ones((16,), jnp.float32)

        z_cp = pltpu.async_copy(stage_v, acc_sh.at[pl.ds(s * _SLICE, _SLICE)],
                                sem_z)
        for cp in e_cps:
            cp.wait()

        is_c0 = c == 0
        for j in range(_CH):
            for i in range(8):
                off = j * 128 + i * 16
                d16 = jnp.where(is_c0, dst1_v[pl.ds(off, 16)],
                                dst2_v[pl.ds(off, 16)])
                s16 = jnp.where(is_c0, src1_v[pl.ds(off, 16)],
                                src2_v[pl.ds(off, 16)])
                flat_v[j, pl.ds(i * 16, 16)] = (
                    (s16 >> 7) * (N * 128) + d16 * 128 + (s16 & 127)
                )

        z_cp.wait()
        plsc.subcore_barrier()
        s_cps = [
            pltpu.async_copy(ones_v, acc_sh.at[flat_v.at[j]], sem_s, add=True)
            for j in range(_CH)
        ]
        for cp in s_cps:
            cp.wait()
        plsc.subcore_barrier()

        pltpu.sync_copy(acc_sh.at[pl.ds(s * _SLICE, _SLICE)],
                        out_hbm.at[c, pl.ds(s * _SLICE, _SLICE)])

    return counts_kernel(ei1, ei2)


def _proj(x, w_ref, b_ref):
    return lax.dot_general(
        x, w_ref[...], (((1,), (0,)), ((), ())),
        preferred_element_type=jnp.float32) + jnp.reshape(b_ref[...], (1, D))


def _proj_t(xt, w_ref, b_ref):
    return lax.dot_general(
        xt, w_ref[...], (((0,), (0,)), ((), ())),
        preferred_element_type=jnp.float32) + jnp.reshape(b_ref[...], (1, D))


def _scores(q, k):
    return lax.dot_general(
        q, k, (((1,), (1,)), ((), ())),
        preferred_element_type=jnp.float32) * _INV_SQRT_D


def _aggregate(e, v):
    return lax.dot_general(
        e, v, (((1,), (0,)), ((), ())), preferred_element_type=jnp.float32)


def _rownorm(y):
    nrm = jnp.sqrt(jnp.sum(y * y, axis=1, keepdims=True))
    return y / jnp.maximum(nrm, 1e-12)


def _tc_a_body(*refs):
    (x1_ref, x2_ref) = refs[:2]
    w_refs = refs[2:10]
    b_refs = refs[10:18]
    s1_ref, s2_ref, v1_ref, sk1_ref, v2_ref, sk2_ref = refs[18:]

    def half(xt, base, s_ref, v_ref, sk_ref):
        q = _proj_t(xt, w_refs[base], b_refs[base])
        k = _proj_t(xt, w_refs[base + 1], b_refs[base + 1])
        v_ref[...] = _proj_t(xt, w_refs[base + 2],
                             b_refs[base + 2]).astype(jnp.bfloat16)
        sk_ref[...] = _proj_t(xt, w_refs[base + 3],
                              b_refs[base + 3]).astype(jnp.bfloat16)
        s_ref[...] = _scores(q, k).astype(jnp.bfloat16)

    half(x1_ref[...], 0, s1_ref, v1_ref, sk1_ref)
    half(x2_ref[...], 4, s2_ref, v2_ref, sk2_ref)


def _tc_b_body(*refs):
    (c_ref, s1_ref, s2_ref, v1_ref, sk1_ref, v2_ref, sk2_ref) = refs[:7]
    w_refs = refs[7:15]
    b_refs = refs[15:23]
    o1_ref, o2_ref = refs[23:]

    def sparse_finish(sc, cnt, v, skip):
        smax = jnp.max(jnp.where(cnt > 0.0, sc, -1e30), axis=1, keepdims=True)
        smax = jnp.where(smax > -1e29, smax, 0.0)
        e = cnt * jnp.exp(jnp.minimum(sc - smax, 50.0))
        den = jnp.sum(e, axis=1, keepdims=True)
        o = _aggregate(e, v) / (den + 1e-16) + skip
        ne = float(N * D)
        mu = jnp.sum(o) / ne
        oc = o - mu
        var = jnp.sum(oc * oc) / ne
        y = oc / jnp.sqrt(var + 1e-5)
        return _rownorm(jnp.maximum(y, 0.0))

    def cross_gat(xa, xb, base):
        q = _proj(xa, w_refs[base], b_refs[base])
        k = _proj(xb, w_refs[base + 1], b_refs[base + 1])
        v = _proj(xb, w_refs[base + 2], b_refs[base + 2])
        sa = _proj(xa, w_refs[base + 3], b_refs[base + 3])
        sb = _proj(xb, w_refs[base + 3], b_refs[base + 3])
        sc = _scores(q, k)
        m = jnp.max(sc, axis=1, keepdims=True)
        e = jnp.exp(sc - m)
        den = jnp.sum(e, axis=1, keepdims=True)
        top = _aggregate(e, v) / (den + 1e-16) + sa
        bot = sb
        ne = float(2 * N * D)
        mu = (jnp.sum(top) + jnp.sum(bot)) / ne
        tc0 = top - mu
        bc0 = bot - mu
        var = (jnp.sum(tc0 * tc0) + jnp.sum(bc0 * bc0)) / ne
        y = tc0 / jnp.sqrt(var + 1e-5)
        y = _rownorm(jnp.maximum(y, 0.0))
        y = _rownorm(y)
        return lax.transpose(y, (1, 0))

    f32 = jnp.float32

    def cnt_panel(g):
        return jnp.concatenate([c_ref[g, 0], c_ref[g, 1], c_ref[g, 2]],
                               axis=1)

    h1 = sparse_finish(s1_ref[...].astype(f32), cnt_panel(0),
                       v1_ref[...].astype(f32), sk1_ref[...].astype(f32))
    h2 = sparse_finish(s2_ref[...].astype(f32), cnt_panel(1),
                       v2_ref[...].astype(f32), sk2_ref[...].astype(f32))
    o1_ref[...] = cross_gat(h1, h2, 0)
    o2_ref[...] = cross_gat(h2, h1, 4)


def _tc_main(x1, x2, cnt, ws, bs):
    f32 = jnp.float32
    bf16 = jnp.bfloat16
    s1, s2, v1, sk1, v2, sk2 = pl.pallas_call(
        _tc_a_body,
        out_shape=(
            jax.ShapeDtypeStruct((N, N), bf16),
            jax.ShapeDtypeStruct((N, N), bf16),
            jax.ShapeDtypeStruct((N, D), bf16),
            jax.ShapeDtypeStruct((N, D), bf16),
            jax.ShapeDtypeStruct((N, D), bf16),
            jax.ShapeDtypeStruct((N, D), bf16),
        ),
    )(x1, x2, *ws[:8], *bs[:8])
    return pl.pallas_call(
        _tc_b_body,
        out_shape=(
            jax.ShapeDtypeStruct((D, N), f32),
            jax.ShapeDtypeStruct((D, N), f32),
        ),
    )(cnt, s1, s2, v1, sk1, v2, sk2, *ws[8:], *bs[8:])


def kernel(x_1, x_2, edge_index_1, edge_index_2, edge_attr_1, edge_attr_2,
           params, place_node_1_idx=0, place_node_2_idx=0):
    order = [(g, t) for g in ("ts", "gs", "tc", "gc") for t in ("q", "k", "v", "s")]
    ws = [params[g]["W" + t] for g, t in order]
    bs = [params[g]["b" + t] for g, t in order]
    cnt = _sc_counts(edge_index_1, edge_index_2)
    o1t, o2t = _tc_main(x_1.T, x_2.T, cnt.reshape(2, 3, N, 128), ws, bs)
    return (o1t.T, o2t.T, 0)


# speedup vs baseline: 1.0286x; 1.0286x over previous
import functools
import math

import jax
import jax.numpy as jnp
from jax import lax
from jax.experimental import pallas as pl
from jax.experimental.pallas import tpu as pltpu
from jax.experimental.pallas import tpu_sc as plsc

N = 384
D = 300
E = 12288
NN = N * N
_INV_SQRT_D = 1.0 / math.sqrt(float(D))

_NSUB = 16
_EPT = E // _NSUB
_CH = _EPT // 128
_SLICE = NN // _NSUB
_ROWS = N // _NSUB


def _sc_counts(ei1, ei2):
    mesh = plsc.VectorSubcoreMesh(core_axis_name="c", subcore_axis_name="s")

    @functools.partial(
        pl.kernel,
        mesh=mesh,
        out_type=jax.ShapeDtypeStruct((2, NN), jnp.float32),
        scratch_types=[
            pltpu.VMEM((_EPT,), jnp.int32),
            pltpu.VMEM((_EPT,), jnp.int32),
            pltpu.VMEM((_EPT,), jnp.int32),
            pltpu.VMEM((_EPT,), jnp.int32),
            pltpu.VMEM((_CH, 128), jnp.int32),
            pltpu.VMEM((128,), jnp.float32),
            pltpu.VMEM((_SLICE,), jnp.float32),
            pltpu.VMEM_SHARED((NN,), jnp.float32),
            pltpu.SemaphoreType.DMA,
            pltpu.SemaphoreType.DMA,
            pltpu.SemaphoreType.DMA,
        ],
    )
    def counts_kernel(ei1_hbm, ei2_hbm, out_hbm, src1_v, dst1_v, src2_v,
                      dst2_v, flat_v, ones_v, stage_v, acc_sh, sem_e, sem_z,
                      sem_s):
        c = lax.axis_index("c")
        s = lax.axis_index("s")
        base = s * _EPT

        e_cps = [
            pltpu.async_copy(ei1_hbm.at[0, pl.ds(base, _EPT)], src1_v, sem_e),
            pltpu.async_copy(ei1_hbm.at[1, pl.ds(base, _EPT)], dst1_v, sem_e),
            pltpu.async_copy(ei2_hbm.at[0, pl.ds(base, _EPT)], src2_v, sem_e),
            pltpu.async_copy(ei2_hbm.at[1, pl.ds(base, _EPT)], dst2_v, sem_e),
        ]

        zeros16 = jnp.zeros((16,), jnp.float32)

        def zero_body(i, carry):
            for u in range(16):
                stage_v[pl.ds(i * 256 + u * 16, 16)] = zeros16
            return carry

        lax.fori_loop(0, _SLICE // 256, zero_body, 0)
        for i in range(8):
            ones_v[pl.ds(i * 16, 16)] = jnp.ones((16,), jnp.float32)

        z_cp = pltpu.async_copy(stage_v, acc_sh.at[pl.ds(s * _SLICE, _SLICE)],
                                sem_z)
        for cp in e_cps:
            cp.wait()

        is_c0 = c == 0
        for j in range(_CH):
            for i in range(8):
                off = j * 128 + i * 16
                f1 = dst1_v[pl.ds(off, 16)] * N + src1_v[pl.ds(off, 16)]
                f2 = dst2_v[pl.ds(off, 16)] * N + src2_v[pl.ds(off, 16)]
                flat_v[j, pl.ds(i * 16, 16)] = jnp.where(is_c0, f1, f2)

        z_cp.wait()
        plsc.subcore_barrier()
        s_cps = [
            pltpu.async_copy(ones_v, acc_sh.at[flat_v.at[j]], sem_s, add=True)
            for j in range(_CH)
        ]
        for cp in s_cps:
            cp.wait()
        plsc.subcore_barrier()

        pltpu.sync_copy(acc_sh.at[pl.ds(s * _SLICE, _SLICE)],
                        out_hbm.at[c, pl.ds(s * _SLICE, _SLICE)])

    return counts_kernel(ei1, ei2)


def _proj(x, w_ref, b_ref):
    return lax.dot_general(
        x, w_ref[...], (((1,), (0,)), ((), ())),
        preferred_element_type=jnp.float32) + jnp.reshape(b_ref[...], (1, D))


def _proj_t(xt, w_ref, b_ref):
    return lax.dot_general(
        xt, w_ref[...], (((0,), (0,)), ((), ())),
        preferred_element_type=jnp.float32) + jnp.reshape(b_ref[...], (1, D))


def _scores(q, k):
    return lax.dot_general(
        q, k, (((1,), (1,)), ((), ())),
        preferred_element_type=jnp.float32) * _INV_SQRT_D


def _aggregate(e, v):
    return lax.dot_general(
        e, v, (((1,), (0,)), ((), ())), preferred_element_type=jnp.float32)


def _rownorm(y):
    nrm = jnp.sqrt(jnp.sum(y * y, axis=1, keepdims=True))
    return y / jnp.maximum(nrm, 1e-12)


def _tc_a_body(*refs):
    (x1_ref, x2_ref) = refs[:2]
    w_refs = refs[2:10]
    b_refs = refs[10:18]
    s1_ref, s2_ref, v1_ref, sk1_ref, v2_ref, sk2_ref = refs[18:]

    def half(xt, base, s_ref, v_ref, sk_ref):
        q = _proj_t(xt, w_refs[base], b_refs[base])
        k = _proj_t(xt, w_refs[base + 1], b_refs[base + 1])
        v_ref[...] = _proj_t(xt, w_refs[base + 2],
                             b_refs[base + 2]).astype(jnp.bfloat16)
        sk_ref[...] = _proj_t(xt, w_refs[base + 3],
                              b_refs[base + 3]).astype(jnp.bfloat16)
        sc = _scores(q, k)
        m = jnp.max(sc, axis=1, keepdims=True)
        s_ref[...] = jnp.exp(sc - m).astype(jnp.bfloat16)

    half(x1_ref[...], 0, s1_ref, v1_ref, sk1_ref)
    half(x2_ref[...], 4, s2_ref, v2_ref, sk2_ref)


def _tc_b_body(*refs):
    (c_ref, s1_ref, s2_ref, v1_ref, sk1_ref, v2_ref, sk2_ref) = refs[:7]
    w_refs = refs[7:15]
    b_refs = refs[15:23]
    o1_ref, o2_ref = refs[23:]

    def sparse_finish(ea, cnt, v, skip):
        e = cnt * ea
        den = jnp.sum(e, axis=1, keepdims=True)
        o = _aggregate(e, v) / (den + 1e-16) + skip
        ne = float(N * D)
        mu = jnp.sum(o) / ne
        oc = o - mu
        var = jnp.sum(oc * oc) / ne
        y = oc / jnp.sqrt(var + 1e-5)
        return _rownorm(jnp.maximum(y, 0.0))

    def cross_gat(xa, xb, base):
        q = _proj(xa, w_refs[base], b_refs[base])
        k = _proj(xb, w_refs[base + 1], b_refs[base + 1])
        v = _proj(xb, w_refs[base + 2], b_refs[base + 2])
        sa = _proj(xa, w_refs[base + 3], b_refs[base + 3])
        sb = _proj(xb, w_refs[base + 3], b_refs[base + 3])
        sc = _scores(q, k)
        m = jnp.max(sc, axis=1, keepdims=True)
        e = jnp.exp(sc - m)
        den = jnp.sum(e, axis=1, keepdims=True)
        top = _aggregate(e, v) / (den + 1e-16) + sa
        bot = sb
        ne = float(2 * N * D)
        mu = (jnp.sum(top) + jnp.sum(bot)) / ne
        tc0 = top - mu
        bc0 = bot - mu
        var = (jnp.sum(tc0 * tc0) + jnp.sum(bc0 * bc0)) / ne
        y = tc0 / jnp.sqrt(var + 1e-5)
        y = _rownorm(jnp.maximum(y, 0.0))
        y = _rownorm(y)
        return lax.transpose(y, (1, 0))

    f32 = jnp.float32
    h1 = sparse_finish(s1_ref[...].astype(f32), c_ref[0].astype(f32),
                       v1_ref[...].astype(f32), sk1_ref[...].astype(f32))
    h2 = sparse_finish(s2_ref[...].astype(f32), c_ref[1].astype(f32),
                       v2_ref[...].astype(f32), sk2_ref[...].astype(f32))
    o1_ref[...] = cross_gat(h1, h2, 0)
    o2_ref[...] = cross_gat(h2, h1, 4)


def _tc_main(x1, x2, cnt, ws, bs):
    f32 = jnp.float32
    bf16 = jnp.bfloat16
    s1, s2, v1, sk1, v2, sk2 = pl.pallas_call(
        _tc_a_body,
        out_shape=(
            jax.ShapeDtypeStruct((N, N), bf16),
            jax.ShapeDtypeStruct((N, N), bf16),
            jax.ShapeDtypeStruct((N, D), bf16),
            jax.ShapeDtypeStruct((N, D), bf16),
            jax.ShapeDtypeStruct((N, D), bf16),
            jax.ShapeDtypeStruct((N, D), bf16),
        ),
    )(x1, x2, *ws[:8], *bs[:8])
    return pl.pallas_call(
        _tc_b_body,
        out_shape=(
            jax.ShapeDtypeStruct((D, N), f32),
            jax.ShapeDtypeStruct((D, N), f32),
        ),
    )(cnt, s1, s2, v1, sk1, v2, sk2, *ws[8:], *bs[8:])


def kernel(x_1, x_2, edge_index_1, edge_index_2, edge_attr_1, edge_attr_2,
           params, place_node_1_idx=0, place_node_2_idx=0):
    order = [(g, t) for g in ("ts", "gs", "tc", "gc") for t in ("q", "k", "v", "s")]
    ws = [params[g]["W" + t] for g, t in order]
    bs = [params[g]["b" + t] for g, t in order]
    cnt = _sc_counts(edge_index_1, edge_index_2)
    o1t, o2t = _tc_main(x_1.T, x_2.T,
                        cnt.reshape(2, N, N).astype(jnp.bfloat16), ws, bs)
    return (o1t.T, o2t.T, 0)
